# baseline (speedup 1.0000x reference)
import jax
import jax.numpy as jnp
from jax import lax
from jax.experimental import pallas as pl
from jax.experimental.pallas import tpu as pltpu
from jax.experimental.pallas import tpu_sc as plsc

NC = 2
NS = 16
NW = NC * NS
SP = 56
KB = 2
NBUF = 4


def _emb_body(table_hbm, idx_hbm, out_hbm, idx_v, bufs, gsem, ssem):
    wid = lax.axis_index("s") * NC + lax.axis_index("c")
    n_b = idx_hbm.shape[0] // (NW * SP)
    n_ch = n_b // KB
    S = out_hbm.shape[1]
    base_e = wid * n_b
    pltpu.sync_copy(idx_hbm.at[pl.ds(base_e * SP, n_b * SP)], idx_v)

    def start_gather(c, b):
        pltpu.make_async_copy(
            table_hbm.at[idx_v.at[pl.ds(c * (KB * SP), KB * SP)]],
            bufs.at[b], gsem).start()

    def wait_gather():
        pltpu.make_async_copy(
            table_hbm.at[pl.ds(0, KB * SP)], bufs.at[0], gsem).wait()

    def wait_scatter():
        pltpu.make_async_copy(
            bufs.at[0, pl.ds(0, S)], out_hbm.at[0], ssem).wait()

    for b in range(NBUF - 1):
        start_gather(b, b)

    @pl.loop(0, n_ch)
    def chunk(c):
        b = lax.rem(c, NBUF)
        wait_gather()
        i0 = base_e + c * KB
        for k in range(KB):
            pltpu.make_async_copy(
                bufs.at[b, pl.ds(k * SP, S)], out_hbm.at[i0 + k], ssem).start()

        @pl.when(c >= 1)
        def _():
            for _k in range(KB):
                wait_scatter()

        @pl.when(c + (NBUF - 1) < n_ch)
        def _():
            start_gather(c + (NBUF - 1), lax.rem(c + (NBUF - 1), NBUF))

    for _k in range(KB):
        wait_scatter()


def kernel(x, W):
    B, S = x.shape
    V, D = W.shape
    idx = jnp.pad(x.astype(jnp.int32), ((0, 0), (0, SP - S))).reshape(-1)
    n_b = B // NW
    mesh = plsc.VectorSubcoreMesh(core_axis_name="c", subcore_axis_name="s")
    run = pl.kernel(
        _emb_body,
        out_type=jax.ShapeDtypeStruct((B, S, D), jnp.float32),
        mesh=mesh,
        compiler_params=pltpu.CompilerParams(use_tc_tiling_on_sc=True),
        scratch_types=[
            pltpu.VMEM((n_b * SP,), jnp.int32),
            pltpu.VMEM((NBUF, KB * SP, D), jnp.float32),
            pltpu.SemaphoreType.DMA,
            pltpu.SemaphoreType.DMA,
        ],
    )
    return run(W, idx)


# speedup vs baseline: 4.2507x; 4.2507x over previous
import jax
import jax.numpy as jnp
from jax import lax
from jax.experimental import pallas as pl
from jax.experimental.pallas import tpu as pltpu
from jax.experimental.pallas import tpu_sc as plsc

NC = 2
NS = 16
NW = NC * NS
LP = 128
CH = 128
NBUF = 4


def _emb_body(table_hbm, idx_hbm, out_hbm, idx_raw, idx_f, bufs, gsem, ssem):
    wid = lax.axis_index("s") * NC + lax.axis_index("c")
    n_e = idx_hbm.shape[0] // NW
    S = idx_f.shape[0] // n_e
    n_ch = (n_e * S) // CH
    base = wid * (n_e * S)
    pltpu.sync_copy(idx_hbm.at[pl.ds(wid * n_e, n_e)], idx_raw)

    @pl.loop(0, n_e)
    def compact(e):
        for k in range(-(-S // 16)):
            idx_f[pl.ds(e * S + k * 16, 16)] = idx_raw[e, pl.ds(k * 16, 16)]

    def start_gather(c, b):
        pltpu.make_async_copy(
            table_hbm.at[idx_f.at[pl.ds(c * CH, CH)]], bufs.at[b], gsem).start()

    def wait_gather():
        pltpu.make_async_copy(
            table_hbm.at[pl.ds(0, CH)], bufs.at[0], gsem).wait()

    def wait_scatter():
        pltpu.make_async_copy(
            bufs.at[0], out_hbm.at[pl.ds(0, CH)], ssem).wait()

    for b in range(NBUF - 1):
        start_gather(b, b)

    @pl.loop(0, n_ch)
    def chunk(c):
        b = lax.rem(c, NBUF)
        wait_gather()
        pltpu.make_async_copy(
            bufs.at[b], out_hbm.at[pl.ds(base + c * CH, CH)], ssem).start()

        @pl.when(c >= 1)
        def _():
            wait_scatter()

        @pl.when(c + (NBUF - 1) < n_ch)
        def _():
            start_gather(c + (NBUF - 1), lax.rem(c + (NBUF - 1), NBUF))

    wait_scatter()


def kernel(x, W):
    B, S = x.shape
    V, D = W.shape
    idx = jnp.pad(x.astype(jnp.int32), ((0, 0), (0, LP - S)), mode="edge")
    n_e = B // NW
    mesh = plsc.VectorSubcoreMesh(core_axis_name="c", subcore_axis_name="s")
    run = pl.kernel(
        _emb_body,
        out_type=jax.ShapeDtypeStruct((B * S, D), jnp.float32),
        mesh=mesh,
        scratch_types=[
            pltpu.VMEM((n_e, LP), jnp.int32),
            pltpu.VMEM((n_e * S + 16,), jnp.int32),
            pltpu.VMEM((NBUF, CH, D), jnp.float32),
            pltpu.SemaphoreType.DMA,
            pltpu.SemaphoreType.DMA,
        ],
    )
    out = run(W, idx)
    return out.reshape(B, S, D)


# speedup vs baseline: 7.0055x; 1.6481x over previous
import jax
import jax.numpy as jnp
from jax import lax
from jax.experimental import pallas as pl
from jax.experimental.pallas import tpu as pltpu
from jax.experimental.pallas import tpu_sc as plsc

NC = 2
NS = 16
NW = NC * NS
LP = 128
CP = 64
NBUF = 6


def _emb_body(table_hbm, idx_hbm, out_hbm, idx_raw, idx_c, bufs, gsem, ssem):
    wid = lax.axis_index("s") * NC + lax.axis_index("c")
    n_e = idx_hbm.shape[0] // NW
    S = out_hbm.shape[1]
    e0 = wid * n_e
    pltpu.sync_copy(idx_hbm.at[pl.ds(e0, n_e)], idx_raw)

    @pl.loop(0, n_e)
    def compact(e):
        for k in range(CP // 16):
            idx_c[pl.ds(e * CP + k * 16, 16)] = idx_raw[e, pl.ds(k * 16, 16)]

    def start_gather(e, b):
        pltpu.make_async_copy(
            table_hbm.at[idx_c.at[pl.ds(e * CP, 56)]],
            bufs.at[b], gsem).start()

    def wait_gather():
        pltpu.make_async_copy(
            table_hbm.at[pl.ds(0, 56)], bufs.at[0], gsem).wait()

    def wait_scatter():
        pltpu.make_async_copy(bufs.at[0, pl.ds(0, S)], out_hbm.at[0], ssem).wait()

    for b in range(NBUF - 1):
        start_gather(b, b)

    @pl.loop(0, n_e)
    def entry(e):
        b = lax.rem(e, NBUF)
        wait_gather()
        pltpu.make_async_copy(bufs.at[b, pl.ds(0, S)], out_hbm.at[e0 + e], ssem).start()

        @pl.when(e >= 1)
        def _():
            wait_scatter()

        @pl.when(e + (NBUF - 1) < n_e)
        def _():
            start_gather(e + (NBUF - 1), lax.rem(e + (NBUF - 1), NBUF))

    wait_scatter()


def kernel(x, W):
    B, S = x.shape
    V, D = W.shape
    idx = jnp.pad(x.astype(jnp.int32), ((0, 0), (0, LP - S)), mode="edge")
    n_e = B // NW
    mesh = plsc.VectorSubcoreMesh(core_axis_name="c", subcore_axis_name="s")
    run = pl.kernel(
        _emb_body,
        out_type=jax.ShapeDtypeStruct((B, S, D), jnp.float32),
        mesh=mesh,
        scratch_types=[
            pltpu.VMEM((n_e, LP), jnp.int32),
            pltpu.VMEM((n_e * CP,), jnp.int32),
            pltpu.VMEM((NBUF, 56, D), jnp.float32),
            pltpu.SemaphoreType.DMA,
            pltpu.SemaphoreType.DMA,
        ],
    )
    return run(W, idx)


# speedup vs baseline: 7.0206x; 1.0022x over previous
import jax
import jax.numpy as jnp
from jax import lax
from jax.experimental import pallas as pl
from jax.experimental.pallas import tpu as pltpu
from jax.experimental.pallas import tpu_sc as plsc

NC = 2
NS = 16
NW = NC * NS
LP = 128
CP = 64
NBUF = 6


def _emb_body(table_hbm, idx_hbm, out_hbm, idx_raw, idx_c, bufs, gsem, ssem):
    wid = lax.axis_index("s") * NC + lax.axis_index("c")
    n_e = idx_hbm.shape[0] // NW
    S = out_hbm.shape[1]
    e0 = wid * n_e
    pltpu.sync_copy(idx_hbm.at[pl.ds(e0, n_e)], idx_raw)

    @pl.loop(0, n_e)
    def compact(e):
        for k in range(CP // 16):
            idx_c[pl.ds(e * CP + k * 16, 16)] = idx_raw[e, pl.ds(k * 16, 16)]

    def start_gather(e, b):
        pltpu.make_async_copy(
            table_hbm.at[idx_c.at[pl.ds(e * CP, 56)]],
            bufs.at[b], gsem).start()

    def wait_gather():
        pltpu.make_async_copy(
            table_hbm.at[pl.ds(0, 56)], bufs.at[0], gsem).wait()

    def wait_scatter():
        pltpu.make_async_copy(bufs.at[0, pl.ds(0, S)], out_hbm.at[0], ssem).wait()

    for b in range(NBUF - 1):
        start_gather(b, b)

    @pl.loop(0, n_e)
    def entry(e):
        b = lax.rem(e, NBUF)
        wait_gather()
        pltpu.make_async_copy(bufs.at[b, pl.ds(0, S)], out_hbm.at[e0 + e], ssem).start()

        @pl.when(e >= 1)
        def _():
            wait_scatter()

        @pl.when(e + (NBUF - 1) < n_e)
        def _():
            start_gather(e + (NBUF - 1), lax.rem(e + (NBUF - 1), NBUF))

    wait_scatter()


def kernel(x, W):
    B, S = x.shape
    V, D = W.shape
    idx = jnp.pad(x.astype(jnp.int32), ((0, 0), (0, LP - S)), mode="edge")
    n_e = B // NW
    mesh = plsc.VectorSubcoreMesh(core_axis_name="c", subcore_axis_name="s")
    run = pl.kernel(
        _emb_body,
        out_type=jax.ShapeDtypeStruct((B, S, D), jnp.float32),
        mesh=mesh,
        compiler_params=pltpu.CompilerParams(use_tc_tiling_on_sc=True),
        scratch_types=[
            pltpu.VMEM((n_e, LP), jnp.int32),
            pltpu.VMEM((n_e * CP,), jnp.int32),
            pltpu.VMEM((NBUF, 56, D), jnp.float32),
            pltpu.SemaphoreType.DMA,
            pltpu.SemaphoreType.DMA,
        ],
    )
    return run(W, idx)
